# baseline (speedup 1.0000x reference)
import functools

import jax
import jax.numpy as jnp
from jax import lax
from jax.experimental import pallas as pl
from jax.experimental.pallas import tpu as pltpu
from jax.experimental.pallas import tpu_sc as plsc

S = 256
F = 2048
NEAR = 0.1
FAR = 100.0
EPS = 1e-8
NCOEF = 32
BAND = 32
NW = 32
BIGF = 1e30


def _coef_body(v_ref, c_ref):
    X0 = v_ref[0]; X1 = v_ref[1]; X2 = v_ref[2]
    Y0 = v_ref[3]; Y1 = v_ref[4]; Y2 = v_ref[5]
    Z0 = v_ref[6]; Z1 = v_ref[7]; Z2 = v_ref[8]
    a0 = Y1 - Y2; b0 = X2 - X1; c0 = X1 * Y2 - X2 * Y1
    a1 = Y2 - Y0; b1 = X0 - X2; c1 = X2 * Y0 - X0 * Y2
    a2 = Y0 - Y1; b2 = X1 - X0; c2 = X0 * Y1 - X1 * Y0
    det = c0 + c1 + c2
    sgn = jnp.where(det >= 0.0, 1.0, -1.0)
    adet = jnp.abs(det)
    valid = adet > EPS
    iZ0 = 1.0 / jnp.where(jnp.abs(Z0) > EPS, Z0, 1.0)
    iZ1 = 1.0 / jnp.where(jnp.abs(Z1) > EPS, Z1, 1.0)
    iZ2 = 1.0 / jnp.where(jnp.abs(Z2) > EPS, Z2, 1.0)
    ga = a0 * iZ0 + a1 * iZ1 + a2 * iZ2
    gb = b0 * iZ0 + b1 * iZ1 + b2 * iZ2
    gc = c0 * iZ0 + c1 * iZ1 + c2 * iZ2
    ymin = jnp.minimum(jnp.minimum(Y0, Y1), Y2)
    ymax = jnp.maximum(jnp.maximum(Y0, Y1), Y2)
    qlo = jnp.clip((S * ymin + (S - 1.0)) * 0.5, -2.0, 300.0)
    qhi = jnp.clip((S * ymax + (S - 1.0)) * 0.5, -2.0, 300.0)
    qlo = jnp.where(valid, qlo, 300.0)
    qhi = jnp.where(valid, qhi, -2.0)
    a0s = a0 * sgn; a1s = a1 * sgn; a2s = a2 * sgn; gas = ga * sgn
    c_ref[0] = a0s
    c_ref[1] = b0 * sgn
    c_ref[2] = c0 * sgn
    c_ref[3] = a1s
    c_ref[4] = b1 * sgn
    c_ref[5] = c1 * sgn
    c_ref[6] = a2s
    c_ref[7] = b2 * sgn
    c_ref[8] = c2 * sgn
    c_ref[9] = gas
    c_ref[10] = gb * sgn
    c_ref[11] = gc * sgn
    c_ref[12] = jnp.where(valid, 1.0 / adet, 0.0)
    c_ref[13] = jnp.where(valid, adet * (1.0 / FAR), BIGF)
    c_ref[14] = jnp.where(valid, adet * (1.0 / NEAR), -BIGF)
    c_ref[15] = qlo
    c_ref[16] = qhi
    c_ref[17] = 1.0 / a0s
    c_ref[18] = 1.0 / a1s
    c_ref[19] = 1.0 / a2s
    c_ref[20] = 1.0 / gas
    z = jnp.zeros_like(det)
    for i in range(21, NCOEF):
        c_ref[i] = z


def _sc_raster_body(coef_hbm, out_hbm, cvm, den, idx):
    cid = lax.axis_index("c")
    sid = lax.axis_index("s")
    wid = sid * 2 + cid
    b = wid // (S // BAND)
    y0 = (wid % (S // BAND)) * BAND

    pltpu.sync_copy(coef_hbm.at[b], cvm)

    def _init(i, _):
        r = i >> 4
        col = pl.multiple_of((i & 15) * 16, 16)
        den[r, pl.ds(col, 16)] = jnp.full((16,), 1.0 / FAR, jnp.float32)
        idx[r, pl.ds(col, 16)] = jnp.full((16,), -1, jnp.int32)
        return 0
    lax.fori_loop(0, BAND * (S // 16), _init, 0)

    lane = lax.iota(jnp.int32, 16)
    y0f = y0.astype(jnp.float32)
    inv_s = jnp.float32(1.0 / S)

    def face_body(f, _):
        v1 = cvm[pl.ds(pl.multiple_of(f * NCOEF, 16), 16)]
        v2 = cvm[pl.ds(pl.multiple_of(f * NCOEF + 16, 16), 16)]
        a0 = v1[0]; b0 = v1[1]; c0 = v1[2]
        a1 = v1[3]; b1 = v1[4]; c1 = v1[5]
        a2 = v1[6]; b2 = v1[7]; c2 = v1[8]
        ga = v1[9]; gb = v1[10]; gc = v1[11]
        radet = v1[12]; glo = v1[13]; ghi = v1[14]
        qlo = v1[15]; qhi = v2[0]
        ra0 = v2[1]; ra1 = v2[2]; ra2 = v2[3]; rga = v2[4]

        rlo = jnp.maximum(qlo.astype(jnp.int32) - 1 - y0, 0)
        rhi = jnp.minimum(qhi.astype(jnp.int32) + 1 - y0, BAND - 1)

        @pl.when(rlo <= rhi)
        def _do_face():
            def row_body(r, _):
                yr = (2.0 * (r.astype(jnp.float32) + y0f) + (1.0 - S)) * inv_s
                b0r = b0 * yr + c0
                b1r = b1 * yr + c1
                b2r = b2 * yr + c2
                bgr = gb * yr + gc
                xlo = jnp.float32(-BIGF)
                xhi = jnp.float32(BIGF)
                for (A, rA, Bc) in (
                        (a0, ra0, b0r),
                        (a1, ra1, b1r),
                        (a2, ra2, b2r),
                        (ga, rga, bgr - glo),
                        (-ga, -rga, ghi - bgr),
                ):
                    t = -Bc * rA
                    xlo = jnp.where(A > 0.0, jnp.maximum(xlo, t), xlo)
                    xhi = jnp.where(A < 0.0, jnp.minimum(xhi, t), xhi)
                    xhi = jnp.where((A == 0.0) & (Bc < 0.0), -BIGF, xhi)
                qjl = jnp.clip((S * xlo + (S - 1.0)) * 0.5, -2.0, 300.0)
                qjh = jnp.clip((S * xhi + (S - 1.0)) * 0.5, -2.0, 300.0)
                jl = jnp.maximum(qjl.astype(jnp.int32) - 1, 0)
                jh = jnp.minimum(qjh.astype(jnp.int32) + 1, S - 1)

                @pl.when(jl <= jh)
                def _do_row():
                    base = jl & (-16)
                    nch = ((jh - base) >> 4) + 1

                    def ch_body(k, _):
                        col = pl.multiple_of(base + k * 16, 16)
                        xv = (2.0 * (lane + col).astype(jnp.float32)
                              + (1.0 - S)) * inv_s
                        w0 = a0 * xv + b0r
                        w1 = a1 * xv + b1r
                        w2 = a2 * xv + b2r
                        g = ga * xv + bgr
                        dn = g * radet
                        dold = den[r, pl.ds(col, 16)]
                        m = ((w0 >= 0.0) & (w1 >= 0.0) & (w2 >= 0.0)
                             & (g > glo) & (g < ghi) & (dn > dold))
                        den[r, pl.ds(col, 16)] = jnp.where(m, dn, dold)
                        iold = idx[r, pl.ds(col, 16)]
                        idx[r, pl.ds(col, 16)] = jnp.where(m, f, iold)
                        return 0

                    lax.fori_loop(0, nch, ch_body, 0)
                return 0

            lax.fori_loop(rlo, rhi + 1, row_body, 0)

        return 0

    lax.fori_loop(0, F, face_body, 0)

    pltpu.sync_copy(idx, out_hbm.at[b, pl.ds(y0, BAND), :])


def kernel(inputs):
    B = inputs.shape[0]
    v = jnp.transpose(inputs, (3, 2, 0, 1)).reshape(9, B, F)
    coef = pl.pallas_call(
        _coef_body,
        out_shape=jax.ShapeDtypeStruct((NCOEF, B, F), jnp.float32),
    )(v)
    coef = jnp.transpose(coef, (1, 2, 0)).reshape(B, F * NCOEF)

    mesh = plsc.VectorSubcoreMesh(core_axis_name="c", subcore_axis_name="s")
    sc = functools.partial(
        pl.kernel,
        mesh=mesh,
        out_type=jax.ShapeDtypeStruct((B, S, S), jnp.int32),
        scratch_types=[
            pltpu.VMEM((F * NCOEF,), jnp.float32),
            pltpu.VMEM((BAND, S), jnp.float32),
            pltpu.VMEM((BAND, S), jnp.int32),
        ],
    )(_sc_raster_body)
    return sc(coef)


# speedup vs baseline: 1.0995x; 1.0995x over previous
import functools

import jax
import jax.numpy as jnp
from jax import lax
from jax.experimental import pallas as pl
from jax.experimental.pallas import tpu as pltpu
from jax.experimental.pallas import tpu_sc as plsc

S = 256
F = 2048
NEAR = 0.1
FAR = 100.0
EPS = 1e-8
NCOEF = 32
BAND = 32
NW = 32
BIGF = 1e30


def _coef_body(v_ref, c_ref):
    X0 = v_ref[0]; X1 = v_ref[1]; X2 = v_ref[2]
    Y0 = v_ref[3]; Y1 = v_ref[4]; Y2 = v_ref[5]
    Z0 = v_ref[6]; Z1 = v_ref[7]; Z2 = v_ref[8]
    a0 = Y1 - Y2; b0 = X2 - X1; c0 = X1 * Y2 - X2 * Y1
    a1 = Y2 - Y0; b1 = X0 - X2; c1 = X2 * Y0 - X0 * Y2
    a2 = Y0 - Y1; b2 = X1 - X0; c2 = X0 * Y1 - X1 * Y0
    det = c0 + c1 + c2
    sgn = jnp.where(det >= 0.0, 1.0, -1.0)
    adet = jnp.abs(det)
    valid = adet > EPS
    iZ0 = 1.0 / jnp.where(jnp.abs(Z0) > EPS, Z0, 1.0)
    iZ1 = 1.0 / jnp.where(jnp.abs(Z1) > EPS, Z1, 1.0)
    iZ2 = 1.0 / jnp.where(jnp.abs(Z2) > EPS, Z2, 1.0)
    ga = a0 * iZ0 + a1 * iZ1 + a2 * iZ2
    gb = b0 * iZ0 + b1 * iZ1 + b2 * iZ2
    gc = c0 * iZ0 + c1 * iZ1 + c2 * iZ2
    ymin = jnp.minimum(jnp.minimum(Y0, Y1), Y2)
    ymax = jnp.maximum(jnp.maximum(Y0, Y1), Y2)
    qlo = jnp.clip((S * ymin + (S - 1.0)) * 0.5, -2.0, 300.0)
    qhi = jnp.clip((S * ymax + (S - 1.0)) * 0.5, -2.0, 300.0)
    qlo = jnp.where(valid, qlo, 300.0)
    qhi = jnp.where(valid, qhi, -2.0)
    a0s = a0 * sgn; a1s = a1 * sgn; a2s = a2 * sgn; gas = ga * sgn
    c_ref[0] = a0s
    c_ref[1] = b0 * sgn
    c_ref[2] = c0 * sgn
    c_ref[3] = a1s
    c_ref[4] = b1 * sgn
    c_ref[5] = c1 * sgn
    c_ref[6] = a2s
    c_ref[7] = b2 * sgn
    c_ref[8] = c2 * sgn
    c_ref[9] = gas
    c_ref[10] = gb * sgn
    c_ref[11] = gc * sgn
    c_ref[12] = jnp.where(valid, 1.0 / adet, 0.0)
    c_ref[13] = jnp.where(valid, adet * (1.0 / FAR), BIGF)
    c_ref[14] = jnp.where(valid, adet * (1.0 / NEAR), -BIGF)
    c_ref[15] = qlo
    c_ref[16] = qhi
    c_ref[17] = 1.0 / a0s
    c_ref[18] = 1.0 / a1s
    c_ref[19] = 1.0 / a2s
    c_ref[20] = 1.0 / gas
    z = jnp.zeros_like(det)
    for i in range(21, NCOEF):
        c_ref[i] = z


def _sc_raster_body(coef_hbm, out_hbm, cvm, den, idx):
    cid = lax.axis_index("c")
    sid = lax.axis_index("s")
    wid = sid * 2 + cid
    b = wid >> 3
    rbase = wid & 7

    pltpu.sync_copy(coef_hbm.at[b], cvm)

    def _init(i, _):
        r = i >> 4
        col = pl.multiple_of((i & 15) * 16, 16)
        den[r, pl.ds(col, 16)] = jnp.full((16,), 1.0 / FAR, jnp.float32)
        idx[r, pl.ds(col, 16)] = jnp.full((16,), -1, jnp.int32)
        return 0
    lax.fori_loop(0, BAND * (S // 16), _init, 0)

    lane = lax.iota(jnp.int32, 16)
    rbase_f = rbase.astype(jnp.float32)
    inv_s = jnp.float32(1.0 / S)

    def face_body(f, _):
        v1 = cvm[pl.ds(pl.multiple_of(f * NCOEF, 16), 16)]
        v2 = cvm[pl.ds(pl.multiple_of(f * NCOEF + 16, 16), 16)]
        a0 = v1[0]; b0 = v1[1]; c0 = v1[2]
        a1 = v1[3]; b1 = v1[4]; c1 = v1[5]
        a2 = v1[6]; b2 = v1[7]; c2 = v1[8]
        ga = v1[9]; gb = v1[10]; gc = v1[11]
        radet = v1[12]; glo = v1[13]; ghi = v1[14]
        qlo = v1[15]; qhi = v2[0]
        ra0 = v2[1]; ra1 = v2[2]; ra2 = v2[3]; rga = v2[4]

        rlo_g = jnp.maximum(qlo.astype(jnp.int32) - 1, 0)
        rhi_g = jnp.minimum(qhi.astype(jnp.int32) + 1, S - 1)
        tlo = jnp.maximum((rlo_g - rbase + 7) >> 3, 0)
        thi = jnp.minimum((rhi_g - rbase) >> 3, BAND - 1)

        @pl.when(tlo <= thi)
        def _do_face():
            def row_body(r, _):
                yr = (2.0 * (rbase_f + 8.0 * r.astype(jnp.float32))
                      + (1.0 - S)) * inv_s
                b0r = b0 * yr + c0
                b1r = b1 * yr + c1
                b2r = b2 * yr + c2
                bgr = gb * yr + gc
                xlo = jnp.float32(-BIGF)
                xhi = jnp.float32(BIGF)
                for (A, rA, Bc) in (
                        (a0, ra0, b0r),
                        (a1, ra1, b1r),
                        (a2, ra2, b2r),
                        (ga, rga, bgr - glo),
                        (-ga, -rga, ghi - bgr),
                ):
                    t = -Bc * rA
                    xlo = jnp.where(A > 0.0, jnp.maximum(xlo, t), xlo)
                    xhi = jnp.where(A < 0.0, jnp.minimum(xhi, t), xhi)
                    xhi = jnp.where((A == 0.0) & (Bc < 0.0), -BIGF, xhi)
                qjl = jnp.clip((S * xlo + (S - 1.0)) * 0.5, -2.0, 300.0)
                qjh = jnp.clip((S * xhi + (S - 1.0)) * 0.5, -2.0, 300.0)
                jl = jnp.maximum(qjl.astype(jnp.int32) - 1, 0)
                jh = jnp.minimum(qjh.astype(jnp.int32) + 1, S - 1)

                @pl.when(jl <= jh)
                def _do_row():
                    base = jl & (-16)
                    nch = ((jh - base) >> 5) + 1

                    def ch_body(k, _):
                        c32 = base + k * 32
                        for h in range(2):
                            col = pl.multiple_of(c32 + h * 16, 16)
                            iv = lane + col
                            xv = (2.0 * iv.astype(jnp.float32)
                                  + (1.0 - S)) * inv_s
                            w0 = a0 * xv + b0r
                            w1 = a1 * xv + b1r
                            w2 = a2 * xv + b2r
                            g = ga * xv + bgr
                            dn = g * radet
                            dold = den[r, pl.ds(col, 16)]
                            m = ((w0 >= 0.0) & (w1 >= 0.0) & (w2 >= 0.0)
                                 & (g > glo) & (g < ghi) & (dn > dold)
                                 & (iv < S))
                            den[r, pl.ds(col, 16)] = jnp.where(m, dn, dold)
                            iold = idx[r, pl.ds(col, 16)]
                            idx[r, pl.ds(col, 16)] = jnp.where(m, f, iold)
                        return 0

                    lax.fori_loop(0, nch, ch_body, 0)
                return 0

            lax.fori_loop(tlo, thi + 1, row_body, 0)

        return 0

    lax.fori_loop(0, F, face_body, 0)

    pltpu.sync_copy(idx.at[pl.ds(0, BAND), :], out_hbm.at[b, rbase])


def kernel(inputs):
    B = inputs.shape[0]
    v = jnp.transpose(inputs, (3, 2, 0, 1)).reshape(9, B, F)
    coef = pl.pallas_call(
        _coef_body,
        out_shape=jax.ShapeDtypeStruct((NCOEF, B, F), jnp.float32),
    )(v)
    coef = jnp.transpose(coef, (1, 2, 0)).reshape(B, F * NCOEF)

    mesh = plsc.VectorSubcoreMesh(core_axis_name="c", subcore_axis_name="s")
    sc = functools.partial(
        pl.kernel,
        mesh=mesh,
        out_type=jax.ShapeDtypeStruct((B, 8, BAND, S), jnp.int32),
        scratch_types=[
            pltpu.VMEM((F * NCOEF,), jnp.float32),
            pltpu.VMEM((BAND + 1, S), jnp.float32),
            pltpu.VMEM((BAND + 1, S), jnp.int32),
        ],
    )(_sc_raster_body)
    out_perm = sc(coef)
    return jnp.transpose(out_perm, (0, 2, 1, 3)).reshape(B, S, S)


# speedup vs baseline: 1.1570x; 1.0523x over previous
import functools

import jax
import jax.numpy as jnp
from jax import lax
from jax.experimental import pallas as pl
from jax.experimental.pallas import tpu as pltpu
from jax.experimental.pallas import tpu_sc as plsc

S = 256
F = 2048
NEAR = 0.1
FAR = 100.0
EPS = 1e-8
NCOEF = 32
BAND = 32
S2 = 272
NW = 32
BIGF = 1e30


def _coef_body(v_ref, c_ref):
    X0 = v_ref[0]; X1 = v_ref[1]; X2 = v_ref[2]
    Y0 = v_ref[3]; Y1 = v_ref[4]; Y2 = v_ref[5]
    Z0 = v_ref[6]; Z1 = v_ref[7]; Z2 = v_ref[8]
    a0 = Y1 - Y2; b0 = X2 - X1; c0 = X1 * Y2 - X2 * Y1
    a1 = Y2 - Y0; b1 = X0 - X2; c1 = X2 * Y0 - X0 * Y2
    a2 = Y0 - Y1; b2 = X1 - X0; c2 = X0 * Y1 - X1 * Y0
    det = c0 + c1 + c2
    sgn = jnp.where(det >= 0.0, 1.0, -1.0)
    adet = jnp.abs(det)
    valid = adet > EPS
    iZ0 = 1.0 / jnp.where(jnp.abs(Z0) > EPS, Z0, 1.0)
    iZ1 = 1.0 / jnp.where(jnp.abs(Z1) > EPS, Z1, 1.0)
    iZ2 = 1.0 / jnp.where(jnp.abs(Z2) > EPS, Z2, 1.0)
    ga = a0 * iZ0 + a1 * iZ1 + a2 * iZ2
    gb = b0 * iZ0 + b1 * iZ1 + b2 * iZ2
    gc = c0 * iZ0 + c1 * iZ1 + c2 * iZ2
    ymin = jnp.minimum(jnp.minimum(Y0, Y1), Y2)
    ymax = jnp.maximum(jnp.maximum(Y0, Y1), Y2)
    qlo = jnp.clip((S * ymin + (S - 1.0)) * 0.5, -2.0, 300.0)
    qhi = jnp.clip((S * ymax + (S - 1.0)) * 0.5, -2.0, 300.0)
    qlo = jnp.where(valid, qlo, 300.0)
    qhi = jnp.where(valid, qhi, -2.0)
    a0s = a0 * sgn; a1s = a1 * sgn; a2s = a2 * sgn; gas = ga * sgn
    c_ref[0] = a0s
    c_ref[1] = b0 * sgn
    c_ref[2] = c0 * sgn
    c_ref[3] = a1s
    c_ref[4] = b1 * sgn
    c_ref[5] = c1 * sgn
    c_ref[6] = a2s
    c_ref[7] = b2 * sgn
    c_ref[8] = c2 * sgn
    c_ref[9] = gas
    c_ref[10] = gb * sgn
    c_ref[11] = gc * sgn
    c_ref[12] = jnp.where(valid, 1.0 / adet, 0.0)
    c_ref[13] = jnp.where(valid, adet * (1.0 / FAR), BIGF)
    c_ref[14] = jnp.where(valid, adet * (1.0 / NEAR), -BIGF)
    c_ref[15] = qlo
    c_ref[16] = qhi
    c_ref[17] = 1.0 / a0s
    c_ref[18] = 1.0 / a1s
    c_ref[19] = 1.0 / a2s
    c_ref[20] = 1.0 / gas
    z = jnp.zeros_like(det)
    for i in range(21, NCOEF):
        c_ref[i] = z


def _sc_raster_body(coef_hbm, out_hbm, cvm, den, idx):
    cid = lax.axis_index("c")
    sid = lax.axis_index("s")
    wid = sid * 2 + cid
    b = wid >> 3
    rbase = wid & 7

    pltpu.sync_copy(coef_hbm.at[b], cvm)

    def _init(r, _):
        for k in range(S2 // 16):
            col = k * 16
            den[r, pl.ds(col, 16)] = jnp.full((16,), 1.0 / FAR, jnp.float32)
            idx[r, pl.ds(col, 16)] = jnp.full((16,), -1, jnp.int32)
        return 0
    lax.fori_loop(0, BAND, _init, 0)

    lane = lax.iota(jnp.int32, 16)
    rbase_f = rbase.astype(jnp.float32)
    inv_s = jnp.float32(1.0 / S)

    def face_body(f, _):
        v1 = cvm[pl.ds(pl.multiple_of(f * NCOEF, 16), 16)]
        v2 = cvm[pl.ds(pl.multiple_of(f * NCOEF + 16, 16), 16)]
        a0 = v1[0]; b0 = v1[1]; c0 = v1[2]
        a1 = v1[3]; b1 = v1[4]; c1 = v1[5]
        a2 = v1[6]; b2 = v1[7]; c2 = v1[8]
        ga = v1[9]; gb = v1[10]; gc = v1[11]
        radet = v1[12]; glo = v1[13]; ghi = v1[14]
        qlo = v1[15]; qhi = v2[0]
        ra0 = v2[1]; ra1 = v2[2]; ra2 = v2[3]; rga = v2[4]

        rlo_g = jnp.maximum(qlo.astype(jnp.int32) - 1, 0)
        rhi_g = jnp.minimum(qhi.astype(jnp.int32) + 1, S - 1)
        tlo = jnp.maximum((rlo_g - rbase + 7) >> 3, 0)
        thi = jnp.minimum((rhi_g - rbase) >> 3, BAND - 1)

        @pl.when(tlo <= thi)
        def _do_face():
            def row_body(r):
                yr = (2.0 * (rbase_f + 8.0 * r.astype(jnp.float32))
                      + (1.0 - S)) * inv_s
                b0r = b0 * yr + c0
                b1r = b1 * yr + c1
                b2r = b2 * yr + c2
                bgr = gb * yr + gc
                xlo = jnp.float32(-BIGF)
                xhi = jnp.float32(BIGF)
                for (A, rA, Bc) in (
                        (a0, ra0, b0r),
                        (a1, ra1, b1r),
                        (a2, ra2, b2r),
                        (ga, rga, bgr - glo),
                        (-ga, -rga, ghi - bgr),
                ):
                    t = -Bc * rA
                    xlo = jnp.where(A > 0.0, jnp.maximum(xlo, t), xlo)
                    xhi = jnp.where(A < 0.0, jnp.minimum(xhi, t), xhi)
                    xhi = jnp.where((A == 0.0) & (Bc < 0.0), -BIGF, xhi)
                qjl = jnp.clip((S * xlo + (S - 1.0)) * 0.5, -2.0, 300.0)
                qjh = jnp.clip((S * xhi + (S - 1.0)) * 0.5, -2.0, 300.0)
                jl = jnp.maximum(qjl.astype(jnp.int32) - 1, 0)
                jh = jnp.minimum(qjh.astype(jnp.int32) + 1, S - 1)

                @pl.when(jl <= jh)
                def _do_row():
                    base = jl & (-16)
                    nch = ((jh - base) >> 5) + 1

                    @plsc.parallel_loop(0, nch)
                    def ch_body(k):
                        c32 = base + k * 32
                        for h in range(2):
                            col = pl.multiple_of(c32 + h * 16, 16)
                            iv = lane + col
                            xv = (2.0 * iv.astype(jnp.float32)
                                  + (1.0 - S)) * inv_s
                            w0 = a0 * xv + b0r
                            w1 = a1 * xv + b1r
                            w2 = a2 * xv + b2r
                            g = ga * xv + bgr
                            dn = g * radet
                            dold = den[r, pl.ds(col, 16)]
                            m = ((w0 >= 0.0) & (w1 >= 0.0) & (w2 >= 0.0)
                                 & (g > glo) & (g < ghi) & (dn > dold))
                            if h == 1:
                                m = m & (iv < S)
                            den[r, pl.ds(col, 16)] = jnp.where(m, dn, dold)
                            iold = idx[r, pl.ds(col, 16)]
                            idx[r, pl.ds(col, 16)] = jnp.where(m, f, iold)

            prow = plsc.parallel_loop(tlo, thi + 1)(row_body)

        return 0

    lax.fori_loop(0, F, face_body, 0)

    pltpu.sync_copy(idx.at[:, pl.ds(0, S)], out_hbm.at[b, rbase])


def kernel(inputs):
    B = inputs.shape[0]
    v = jnp.transpose(inputs, (3, 2, 0, 1)).reshape(9, B, F)
    coef = pl.pallas_call(
        _coef_body,
        out_shape=jax.ShapeDtypeStruct((NCOEF, B, F), jnp.float32),
    )(v)
    coef = jnp.transpose(coef, (1, 2, 0)).reshape(B, F * NCOEF)

    mesh = plsc.VectorSubcoreMesh(core_axis_name="c", subcore_axis_name="s")
    sc = functools.partial(
        pl.kernel,
        mesh=mesh,
        out_type=jax.ShapeDtypeStruct((B, 8, BAND, S), jnp.int32),
        scratch_types=[
            pltpu.VMEM((F * NCOEF,), jnp.float32),
            pltpu.VMEM((BAND, S2), jnp.float32),
            pltpu.VMEM((BAND, S2), jnp.int32),
        ],
    )(_sc_raster_body)
    out_perm = sc(coef)
    return jnp.transpose(out_perm, (0, 2, 1, 3)).reshape(B, S, S)
